# baseline (speedup 1.0000x reference)
import jax
import jax.numpy as jnp
from jax import lax
from jax.experimental import pallas as pl
from jax.experimental.pallas import tpu as pltpu
from jax.experimental.pallas import tpu_sc as plsc

NC = 2
NS = 16
NW = NC * NS
LP = 128
CP = 64
NBUF = 8


def _emb_body(table_hbm, idx_hbm, out_hbm, idx_raw, idx_c, bufs, gsem, ssem):
    wid = lax.axis_index("s") * NC + lax.axis_index("c")
    n_e = idx_hbm.shape[0] // NW
    S = out_hbm.shape[1]
    e0 = wid * n_e
    pltpu.sync_copy(idx_hbm.at[pl.ds(e0, n_e)], idx_raw)

    @pl.loop(0, n_e)
    def compact(e):
        for k in range(CP // 16):
            idx_c[pl.ds(e * CP + k * 16, 16)] = idx_raw[e, pl.ds(k * 16, 16)]

    def start_gather(e, b):
        pltpu.make_async_copy(
            table_hbm.at[idx_c.at[pl.ds(e * CP, 56)]],
            bufs.at[b], gsem).start()

    def wait_gather():
        pltpu.make_async_copy(
            table_hbm.at[pl.ds(0, 56)], bufs.at[0], gsem).wait()

    def wait_scatter():
        pltpu.make_async_copy(bufs.at[0, pl.ds(0, S)], out_hbm.at[0], ssem).wait()

    for b in range(NBUF - 2):
        start_gather(b, b)

    @pl.loop(0, n_e)
    def entry(e):
        b = lax.rem(e, NBUF)
        wait_gather()
        pltpu.make_async_copy(bufs.at[b, pl.ds(0, S)], out_hbm.at[e0 + e], ssem).start()

        @pl.when(e >= 2)
        def _():
            wait_scatter()

        @pl.when(e + (NBUF - 2) < n_e)
        def _():
            start_gather(e + (NBUF - 2), lax.rem(e + (NBUF - 2), NBUF))

    wait_scatter()
    wait_scatter()


def kernel(x, W):
    B, S = x.shape
    V, D = W.shape
    idx = jnp.pad(x.astype(jnp.int32), ((0, 0), (0, LP - S)), mode="edge")
    n_e = B // NW
    mesh = plsc.VectorSubcoreMesh(core_axis_name="c", subcore_axis_name="s")
    run = pl.kernel(
        _emb_body,
        out_type=jax.ShapeDtypeStruct((B, S, D), jnp.float32),
        mesh=mesh,
        scratch_types=[
            pltpu.VMEM((n_e, LP), jnp.int32),
            pltpu.VMEM((n_e * CP,), jnp.int32),
            pltpu.VMEM((NBUF, 56, D), jnp.float32),
            pltpu.SemaphoreType.DMA,
            pltpu.SemaphoreType.DMA,
        ],
    )
    return run(W, idx)


# speedup vs baseline: 1.0010x; 1.0010x over previous
import jax
import jax.numpy as jnp
from jax import lax
from jax.experimental import pallas as pl
from jax.experimental.pallas import tpu as pltpu
from jax.experimental.pallas import tpu_sc as plsc

NC = 2
NS = 16
NW = NC * NS
LP = 128
CP = 64
NBUF = 8


def _emb_body(table_hbm, idx_hbm, out_hbm, idx_raw, idx_c, bufs, gsem, ssem):
    wid = lax.axis_index("s") * NC + lax.axis_index("c")
    n_e = idx_hbm.shape[0] // NW
    S = out_hbm.shape[1]
    e0 = wid * n_e
    pltpu.sync_copy(idx_hbm.at[pl.ds(e0, n_e)], idx_raw)

    @pl.loop(0, n_e)
    def compact(e):
        for k in range(CP // 16):
            idx_c[pl.ds(e * CP + k * 16, 16)] = idx_raw[e, pl.ds(k * 16, 16)]

    def start_gather(e, b):
        pltpu.make_async_copy(
            table_hbm.at[idx_c.at[pl.ds(e * CP, 56)]],
            bufs.at[b], gsem).start()

    def wait_gather():
        pltpu.make_async_copy(
            table_hbm.at[pl.ds(0, 56)], bufs.at[0], gsem).wait()

    def wait_scatter():
        pltpu.make_async_copy(bufs.at[0, pl.ds(0, S)], out_hbm.at[0], ssem).wait()

    for b in range(NBUF - 1):
        start_gather(b, b)

    @pl.loop(0, n_e)
    def entry(e):
        b = lax.rem(e, NBUF)
        wait_gather()
        pltpu.make_async_copy(bufs.at[b, pl.ds(0, S)], out_hbm.at[e0 + e], ssem).start()

        @pl.when(e >= 1)
        def _():
            wait_scatter()

        @pl.when(e + (NBUF - 1) < n_e)
        def _():
            start_gather(e + (NBUF - 1), lax.rem(e + (NBUF - 1), NBUF))

    wait_scatter()


def kernel(x, W):
    B, S = x.shape
    V, D = W.shape
    idx = jnp.pad(x.astype(jnp.int32), ((0, 0), (0, LP - S)), mode="edge")
    n_e = B // NW
    mesh = plsc.VectorSubcoreMesh(core_axis_name="c", subcore_axis_name="s")
    run = pl.kernel(
        _emb_body,
        out_type=jax.ShapeDtypeStruct((B, S, D), jnp.float32),
        mesh=mesh,
        scratch_types=[
            pltpu.VMEM((n_e, LP), jnp.int32),
            pltpu.VMEM((n_e * CP,), jnp.int32),
            pltpu.VMEM((NBUF, 56, D), jnp.float32),
            pltpu.SemaphoreType.DMA,
            pltpu.SemaphoreType.DMA,
        ],
    )
    return run(W, idx)
